# Initial kernel scaffold; baseline (speedup 1.0000x reference)
import jax
import jax.numpy as jnp
from jax.experimental import pallas as pl


def kernel(particles, u, z, M, Q_val, key):
    raise NotImplementedError("write your pallas kernel here")


# speedup vs baseline: 25.1257x; 25.1257x over previous
import functools

import jax
import jax.numpy as jnp
from jax import lax
from jax.experimental import pallas as pl
from jax.experimental.pallas import tpu as pltpu
from jax.experimental.pallas import tpu_sc as plsc

N = 1048576
R = 8192
C = 128
K = 64
DT = 0.1
INV_N = 1.0 / N


def _ew_body(x0, y0, th0, n0, n1, sc, px_o, py_o, th_o, w_o):
    u0 = sc[0, 0] + n0[...]
    u1 = sc[0, 1] + n1[...]
    z = sc[0, 2]
    q = sc[0, 3]
    th_old = th0[...]
    px = x0[...] + u0 * jnp.cos(th_old) * DT
    py = y0[...] + u0 * jnp.sin(th_old) * DT
    z_exp = jnp.sqrt(px ** 2 + py ** 2)
    innov = z - z_exp
    w_o[...] = jnp.exp(-0.5 * innov ** 2 / q) + 1e-08
    px_o[...] = px
    py_o[...] = py
    th_o[...] = th_old + u1 * DT


def _elementwise(x0, y0, th0, n0, n1, sc):
    bs = pl.BlockSpec((1024, C), lambda i: (i, 0))
    return pl.pallas_call(
        _ew_body,
        grid=(R // 1024,),
        in_specs=[bs, bs, bs, bs, bs, pl.BlockSpec((1, 4), lambda i: (0, 0))],
        out_specs=[bs, bs, bs, bs],
        out_shape=[jax.ShapeDtypeStruct((R, C), jnp.float32)] * 4,
    )(x0, y0, th0, n0, n1, sc)


def _hier_scan(x, t_ref, r3_ref, off3_ref, off_ref, zero):
    t_ref[...] = x.T
    for i in range(1, C):
        t_ref[i:i + 1, :] = t_ref[i - 1:i, :] + t_ref[i:i + 1, :]
    bs = t_ref[C - 1:C, :]
    for k in range(K):
        r3_ref[:, k:k + 1] = bs[:, k * C:(k + 1) * C].T
    for i in range(1, C):
        r3_ref[i:i + 1, :] = r3_ref[i - 1:i, :] + r3_ref[i:i + 1, :]
    off3_ref[0:1, 0:1] = jnp.full((1, 1), zero, r3_ref.dtype)
    for k in range(1, K):
        off3_ref[0:1, k:k + 1] = (off3_ref[0:1, k - 1:k]
                                  + r3_ref[C - 1:C, k - 1:k])
    r3b = r3_ref[...] + off3_ref[...]
    row0 = jnp.concatenate([jnp.full((1, 1), zero, r3b.dtype),
                            r3b[C - 1:C, :K - 1]], axis=1)
    ex = jnp.concatenate([row0, r3b[:C - 1, :]], axis=0)
    ex_t = ex.T
    for k in range(K):
        off_ref[0:1, k * C:(k + 1) * C] = ex_t[k:k + 1, :]
    return (t_ref[...] + off_ref[...]).T


def _cumsum_body(w_ref, rs_ref, c_o, t_ref, r3_ref, off3_ref, off_ref):
    wn = w_ref[...] * rs_ref[0, 0]
    c_o[...] = _hier_scan(wn, t_ref, r3_ref, off3_ref, off_ref, jnp.float32(0.0))


def _cumsum(w, rs):
    return pl.pallas_call(
        _cumsum_body,
        in_specs=[pl.BlockSpec(memory_space=pltpu.VMEM),
                  pl.BlockSpec(memory_space=pltpu.SMEM)],
        out_specs=pl.BlockSpec(memory_space=pltpu.VMEM),
        out_shape=jax.ShapeDtypeStruct((R, C), jnp.float32),
        scratch_shapes=[pltpu.VMEM((C, R), jnp.float32),
                        pltpu.VMEM((C, K), jnp.float32),
                        pltpu.VMEM((1, K), jnp.float32),
                        pltpu.VMEM((1, R), jnp.float32)],
    )(w, rs)


def _icumsum_body(h_ref, idx_o, t_ref, r3_ref, off3_ref, off_ref):
    idx_o[...] = _hier_scan(h_ref[...], t_ref, r3_ref, off3_ref, off_ref,
                            jnp.int32(0))


def _icumsum(h):
    return pl.pallas_call(
        _icumsum_body,
        in_specs=[pl.BlockSpec(memory_space=pltpu.VMEM)],
        out_specs=pl.BlockSpec(memory_space=pltpu.VMEM),
        out_shape=jax.ShapeDtypeStruct((R, C), jnp.int32),
        scratch_shapes=[pltpu.VMEM((C, R), jnp.int32),
                        pltpu.VMEM((C, K), jnp.int32),
                        pltpu.VMEM((1, K), jnp.int32),
                        pltpu.VMEM((1, R), jnp.int32)],
    )(h)


def _bounds_body(c_ref, px_ref, py_ref, th_ref, sc_ref, gv_o, sums_o):
    cc = c_ref[...]
    rr = sc_ref[0, 0]
    nf = jnp.float32(N)
    t = (cc - rr) * nf
    g = jnp.clip(jnp.floor(t), -1.0, N - 1).astype(jnp.int32)
    inv_n = jnp.float32(INV_N)
    for _ in range(3):
        cand = jnp.clip(g + 1, 0, N - 1)
        uv = rr + cand.astype(jnp.float32) * inv_n
        ok = (uv <= cc) & (g + 1 <= N - 1)
        g = jnp.where(ok, g + 1, g)
    for _ in range(3):
        uv = rr + jnp.clip(g, 0, N - 1).astype(jnp.float32) * inv_n
        bad = (g >= 0) & (uv > cc)
        g = jnp.where(bad, g - 1, g)
    gv = g + 1
    last_row = jnp.concatenate(
        [gv[R - 1:R, :C - 1], jnp.full((1, 1), N, jnp.int32)], axis=1)
    gvv = jnp.concatenate([gv[:R - 1, :], last_row], axis=0)
    col0 = jnp.concatenate([jnp.zeros((1, 1), jnp.int32),
                            gvv[:R - 1, C - 1:C]], axis=0)
    prev = jnp.concatenate([col0, gvv[:, :C - 1]], axis=1)
    cnt = (gvv - prev).astype(jnp.float32)
    th = th_ref[...]
    sums_o[0, 0] = jnp.sum(cnt * px_ref[...])
    sums_o[0, 1] = jnp.sum(cnt * py_ref[...])
    sums_o[0, 2] = jnp.sum(cnt * jnp.sin(th))
    sums_o[0, 3] = jnp.sum(cnt * jnp.cos(th))
    gv_o[...] = gvv


def _bounds(c, px, py, th, sc):
    return pl.pallas_call(
        _bounds_body,
        in_specs=[pl.BlockSpec(memory_space=pltpu.VMEM)] * 4
        + [pl.BlockSpec(memory_space=pltpu.SMEM)],
        out_specs=[pl.BlockSpec(memory_space=pltpu.VMEM),
                   pl.BlockSpec(memory_space=pltpu.SMEM)],
        out_shape=[jax.ShapeDtypeStruct((R, C), jnp.int32),
                   jax.ShapeDtypeStruct((1, 4), jnp.float32)],
    )(c, px, py, th, sc)


NH = N // 2
HPAD = NH + 256
_SC_NW = 32
_CH = N // _SC_NW
_HS = HPAD // _SC_NW
_OH = NH // _SC_NW


def _hist_sc(gv_flat, ones_hbm, zeros_hbm):
    mesh = plsc.VectorSubcoreMesh(core_axis_name="c", subcore_axis_name="s")

    @functools.partial(
        pl.kernel, mesh=mesh,
        out_type=jax.ShapeDtypeStruct((NH,), jnp.int32),
        scratch_types=[pltpu.VMEM((_CH,), jnp.int32),
                       pltpu.VMEM((_CH,), jnp.int32),
                       pltpu.VMEM((_HS,), jnp.int32),
                       pltpu.VMEM_SHARED((HPAD,), jnp.int32)],
    )
    def hist_kernel(gv_hbm, ones_in, zeros_in, hist_out, idx_v, ones_v, z_v,
                    shist):
        wid = lax.axis_index("s") * 2 + lax.axis_index("c")
        base = wid * _CH
        pltpu.sync_copy(zeros_in.at[pl.ds(wid * _HS, _HS)], z_v)
        pltpu.sync_copy(z_v, shist.at[pl.ds(wid * _HS, _HS)])
        pltpu.sync_copy(gv_hbm.at[pl.ds(base, _CH)], idx_v)
        pltpu.sync_copy(ones_in.at[pl.ds(base, _CH)], ones_v)
        plsc.subcore_barrier()
        pltpu.sync_copy(ones_v, shist.at[idx_v], add=True)
        plsc.subcore_barrier()
        pltpu.sync_copy(shist.at[pl.ds(wid * _OH, _OH)], z_v.at[pl.ds(0, _OH)])
        pltpu.sync_copy(z_v.at[pl.ds(0, _OH)], hist_out.at[pl.ds(wid * _OH, _OH)])

    return hist_kernel(gv_flat, ones_hbm, zeros_hbm)


_GSUB = 8192


def _gather_sc(table, idx_flat):
    mesh = plsc.VectorSubcoreMesh(core_axis_name="c", subcore_axis_name="s")

    @functools.partial(
        pl.kernel, mesh=mesh,
        compiler_params=pltpu.CompilerParams(use_tc_tiling_on_sc=False),
        out_type=jax.ShapeDtypeStruct((N, 3), jnp.float32),
        scratch_types=[pltpu.VMEM((_CH,), jnp.int32),
                       pltpu.VMEM((_GSUB, 3), jnp.float32),
                       pltpu.SemaphoreType.DMA],
    )
    def gather_kernel(table_hbm, idx_hbm, out_hbm, idx_v, rows_v, sem):
        wid = lax.axis_index("s") * 2 + lax.axis_index("c")
        base = wid * _CH
        pltpu.sync_copy(idx_hbm.at[pl.ds(base, _CH)], idx_v)
        for j in range(_CH // _GSUB):
            pltpu.async_copy(
                table_hbm.at[idx_v.at[pl.ds(j * _GSUB, _GSUB)]], rows_v,
                sem).wait()
            pltpu.sync_copy(rows_v,
                            out_hbm.at[pl.ds(base + j * _GSUB, _GSUB)])

    return gather_kernel(table, idx_flat)


def kernel(particles, u, z, M, Q_val, key):
    key_motion, key_resample = jax.random.split(key)
    u_noise = jax.random.multivariate_normal(
        key_motion, jnp.zeros_like(u), M, shape=(N,))
    rr = jax.random.uniform(key_resample, minval=0.0, maxval=1.0 / N)

    x0 = particles[:, 0].reshape(R, C)
    y0 = particles[:, 1].reshape(R, C)
    th0 = particles[:, 2].reshape(R, C)
    n0 = u_noise[:, 0].reshape(R, C)
    n1 = u_noise[:, 1].reshape(R, C)
    sc = jnp.stack([u[0], u[1], z[0], Q_val]).reshape(1, 4)

    px, py, th, w = _elementwise(x0, y0, th0, n0, n1, sc)

    s_total = jnp.sum(w.reshape(-1))
    rs = (jnp.float32(1.0) / s_total).reshape(1, 1)
    c = _cumsum(w, rs)

    sc2 = rr.reshape(1, 1)
    gv, sums = _bounds(c, px, py, th, sc2)

    gvf = gv.reshape(-1)
    hist = jnp.zeros((N + 1,), jnp.int32).at[gvf].add(1)[:N]

    idx = _icumsum(hist.reshape(R, C))

    table = jnp.stack([px.reshape(-1), py.reshape(-1), th.reshape(-1)], axis=1)
    resampled = jnp.take(table, idx.reshape(-1), axis=0)

    nf = jnp.float32(N)
    mu = jnp.array([sums[0, 0] / nf, sums[0, 1] / nf,
                    jnp.arctan2(sums[0, 2] / nf, sums[0, 3] / nf)])
    return (mu, resampled)
